# baseline (speedup 1.0000x reference)
import functools

import numpy as np
import jax
import jax.numpy as jnp
from jax import lax
from jax.experimental import pallas as pl
from jax.experimental.pallas import tpu as pltpu
from jax.experimental.pallas import tpu_sc as plsc

_NL = 19
_FD = 2
_BASE_RES = 16
_GROW = 1.5
_T = 1 << 24
_PRIME1 = np.int32(2654435761 - (1 << 32))
_MASK24 = np.int32(_T - 1)

_LEVELS = []
_off_acc = 0
for _l in range(_NL):
    _s = _BASE_RES * (_GROW ** _l) - 1.0
    _r = int(np.ceil(_s)) + 1
    _sz = min(_r * _r, _T)
    _LEVELS.append((np.float32(_s), _r, _sz, _off_acc, _sz == _r * _r))
    _off_acc += _sz

_NC = 2
_NS = 16
_L = 16
_NW = _NC * _NS
_B = 1024
_DIN = _NL * _FD


def _sc_encode(pos_flat, tab_flat):
    n = pos_flat.shape[0] // 2
    ppw = n // _NW
    nblk = ppw // _B
    mesh = plsc.VectorSubcoreMesh(core_axis_name="c", subcore_axis_name="s")

    @functools.partial(
        pl.kernel,
        mesh=mesh,
        out_type=jax.ShapeDtypeStruct((n * _DIN,), jnp.float32),
        compiler_params=pltpu.CompilerParams(needs_layout_passes=False),
        scratch_types=[
            pltpu.VMEM((2 * _B,), jnp.float32),
            pltpu.VMEM((_B,), jnp.float32),
            pltpu.VMEM((_B,), jnp.float32),
            [pltpu.VMEM((_B,), jnp.int32) for _ in range(4)],
            [pltpu.VMEM((_B,), jnp.int32) for _ in range(4)],
            pltpu.VMEM((4 * _B,), jnp.float32),
            [pltpu.VMEM((_B,), jnp.float32) for _ in range(4)],
            [pltpu.VMEM((_B,), jnp.float32) for _ in range(4)],
            pltpu.VMEM((_B * _DIN,), jnp.float32),
            pltpu.SemaphoreType.DMA,
        ],
    )
    def enc(pos_hbm, tab_hbm, out_hbm, pos_v, px_v, py_v,
            ix_refs, iy_refs, w_v, rx_refs, ry_refs, feat_v, sem):
        wid = lax.axis_index("s") * _NC + lax.axis_index("c")
        lane = lax.iota(jnp.int32, _L)
        nvr = _B // _L

        def block_body(blk, carry):
            base_pt = (wid * nblk + blk) * _B
            pltpu.sync_copy(pos_hbm.at[pl.ds(2 * base_pt, 2 * _B)], pos_v)

            def deint(i, c):
                r2 = 2 * (i * _L + lane)
                xs = plsc.load_gather(pos_v, [r2])
                ys = plsc.load_gather(pos_v, [r2 + 1])
                px_v[pl.ds(i * _L, _L)] = (xs - 0.5) * 2.0
                py_v[pl.ds(i * _L, _L)] = (ys - 0.5) * 2.0
                return c
            lax.fori_loop(0, nvr, deint, 0, unroll=False)

            for li, (scale, res, size, off, dense) in enumerate(_LEVELS):
                def iw(i, c, scale=scale, res=res, off=off, dense=dense):
                    b = i * _L
                    xv = px_v[pl.ds(b, _L)] * scale + 0.5
                    yv = py_v[pl.ds(b, _L)] * scale + 0.5
                    ix = xv.astype(jnp.int32)
                    ix = jnp.where(ix.astype(jnp.float32) > xv, ix - 1, ix)
                    iy = yv.astype(jnp.int32)
                    iy = jnp.where(iy.astype(jnp.float32) > yv, iy - 1, iy)
                    wx = xv - ix.astype(jnp.float32)
                    wy = yv - iy.astype(jnp.float32)
                    for ci, (dx, dy) in enumerate(((0, 0), (0, 1), (1, 0), (1, 1))):
                        cx = ix + dx
                        cy = iy + dy
                        if dense:
                            cxc = jnp.clip(cx, 0, res - 1)
                            cyc = jnp.clip(cy, 0, res - 1)
                            iv = cyc * res + cxc + off
                        else:
                            iv = (cx ^ (cy * _PRIME1)) & _MASK24
                            iv = iv + off
                        iv2 = iv * 2
                        wv = (wx if dx else (1.0 - wx)) * (wy if dy else (1.0 - wy))
                        ix_refs[ci][pl.ds(b, _L)] = iv2
                        iy_refs[ci][pl.ds(b, _L)] = iv2 + 1
                        w_v[pl.ds(ci * _B + b, _L)] = wv
                    return c
                lax.fori_loop(0, nvr, iw, 0, unroll=False)

                cps = [pltpu.async_copy(tab_hbm.at[ix_refs[ci]], rx_refs[ci], sem)
                       for ci in range(4)]
                cps += [pltpu.async_copy(tab_hbm.at[iy_refs[ci]], ry_refs[ci], sem)
                        for ci in range(4)]
                for cp in cps:
                    cp.wait()

                def accum(i, c, li=li):
                    b = i * _L
                    r = b + lane
                    ax = jnp.zeros((_L,), jnp.float32)
                    ay = jnp.zeros((_L,), jnp.float32)
                    for ci in range(4):
                        wv = w_v[pl.ds(ci * _B + b, _L)]
                        ax = ax + rx_refs[ci][pl.ds(b, _L)] * wv
                        ay = ay + ry_refs[ci][pl.ds(b, _L)] * wv
                    fbase = r * _DIN + (2 * li)
                    plsc.store_scatter(feat_v, [fbase], ax)
                    plsc.store_scatter(feat_v, [fbase + 1], ay)
                    return c
                lax.fori_loop(0, nvr, accum, 0, unroll=False)

            pltpu.sync_copy(feat_v, out_hbm.at[pl.ds(base_pt * _DIN, _B * _DIN)])
            return carry

        lax.fori_loop(0, nblk, block_body, 0, unroll=False)

    return enc(pos_flat, tab_flat)


def _mlp(feats, w1, w2, w3):
    n, d_in = feats.shape
    bn = 1024

    def body(g_ref, w1_ref, w2_ref, w3_ref, o_ref):
        g = g_ref[...]
        h = jnp.maximum(jnp.dot(g, w1_ref[...], preferred_element_type=jnp.float32), 0.0)
        h = jnp.maximum(jnp.dot(h, w2_ref[...], preferred_element_type=jnp.float32), 0.0)
        o = jnp.dot(h, w3_ref[...], preferred_element_type=jnp.float32)
        o_ref[...] = jnp.clip(o, -1.0, 1.0)

    return pl.pallas_call(
        body,
        grid=(n // bn,),
        in_specs=[
            pl.BlockSpec((bn, d_in), lambda i: (i, 0)),
            pl.BlockSpec(w1.shape, lambda i: (0, 0)),
            pl.BlockSpec(w2.shape, lambda i: (0, 0)),
            pl.BlockSpec(w3.shape, lambda i: (0, 0)),
        ],
        out_specs=pl.BlockSpec((bn, 2), lambda i: (i, 0)),
        out_shape=jax.ShapeDtypeStruct((n, 2), jnp.float32),
    )(feats, w1, w2, w3)


def kernel(in_pos, table, W1, W2, W3):
    n = in_pos.shape[0]
    feats_flat = _sc_encode(in_pos.reshape(-1), table.reshape(-1))
    feats = feats_flat.reshape(n, _DIN)
    return _mlp(feats, W1, W2, W3)
